# baseline (speedup 1.0000x reference)
import functools

import jax
import jax.numpy as jnp
from jax import lax
from jax.experimental import pallas as pl
from jax.experimental.pallas import tpu as pltpu
from jax.experimental.pallas import tpu_sc as plsc

B = 16384
D = 16
NC = 2
NS = 16
L = 16
NW = NC * NS
BPW = B // NW
IPW = 2 * BPW
ICHUNK = 128
NCHUNK = IPW // ICHUNK
GROUPS = BPW // L


def _lfm_body(x_hbm, table_hbm, out_hbm, idx_v, rows_v, out_v, sem):
    wid = lax.axis_index("s") * NC + lax.axis_index("c")
    base = wid * BPW

    pltpu.sync_copy(x_hbm.at[pl.ds(wid * NCHUNK, NCHUNK)], idx_v)

    copies = [
        pltpu.async_copy(
            table_hbm.at[idx_v.at[j]],
            rows_v.at[pl.ds(j * ICHUNK, ICHUNK)],
            sem,
        )
        for j in range(NCHUNK)
    ]
    for c in copies:
        c.wait()

    lanes = lax.iota(jnp.int32, L)

    def group(g, _):
        r0 = 2 * L * g + 2 * lanes
        r1 = r0 + 1
        acc = jnp.zeros((L,), jnp.float32)
        for d in range(D):
            dvec = jnp.full((L,), d, jnp.int32)
            a = plsc.load_gather(rows_v, [r0, dvec])
            b = plsc.load_gather(rows_v, [r1, dvec])
            acc = acc + a * b
        out_v[pl.ds(g * L, L)] = 1.0 / (1.0 + jnp.exp(-acc))
        return 0

    lax.fori_loop(0, GROUPS, group, 0)

    pltpu.sync_copy(out_v, out_hbm.at[pl.ds(base, BPW)])


@functools.partial(
    pl.kernel,
    out_type=jax.ShapeDtypeStruct((B,), jnp.float32),
    mesh=plsc.VectorSubcoreMesh(core_axis_name="c", subcore_axis_name="s"),
    compiler_params=pltpu.CompilerParams(
        needs_layout_passes=False, use_tc_tiling_on_sc=False
    ),
    scratch_types=[
        pltpu.VMEM((NCHUNK, ICHUNK), jnp.int32),
        pltpu.VMEM((IPW, D), jnp.float32),
        pltpu.VMEM((BPW,), jnp.float32),
        pltpu.SemaphoreType.DMA,
    ],
)
def _lfm_sc(x_hbm, table_hbm, out_hbm, idx_v, rows_v, out_v, sem):
    _lfm_body(x_hbm, table_hbm, out_hbm, idx_v, rows_v, out_v, sem)


def kernel(x, table):
    x2 = x.astype(jnp.int32).reshape(NW * NCHUNK, ICHUNK)
    out = _lfm_sc(x2, table)
    return out.reshape(B, 1)
